# baseline (speedup 1.0000x reference)
import functools

import numpy as np
import jax
import jax.numpy as jnp
from jax import lax
from jax.experimental import pallas as pl
from jax.experimental.pallas import tpu as pltpu
from jax.experimental.pallas import tpu_sc as plsc

_SPATIAL = ((128, 128), (64, 64), (32, 32))
_NH = 8
_HD = 32
_Q = 21504
_NCOL = 96
_NPAD = 128
_BQ = 512

_NW = 32
_QPW = _Q // _NW
_QB = 48
_NCHUNK = _QPW // _QB


def _build_consts():
    starts = (0, 16384, 20480)
    cW = np.ones((1, _NPAD), np.float32)
    cH = np.ones((1, _NPAD), np.float32)
    cS = np.zeros((1, _NPAD), np.float32)
    cHd = np.zeros((1, _NPAD), np.float32)
    E3 = np.zeros((3, _NPAD), np.float32)
    HS = np.zeros((_NPAD, _NPAD), np.float32)
    MSK = np.zeros((1, _NPAD), np.float32)
    for j in range(_NCOL):
        h = j // 12
        l = (j % 12) // 4
        hh, ww = _SPATIAL[l]
        cW[0, j] = ww
        cH[0, j] = hh
        cS[0, j] = starts[l]
        cHd[0, j] = h
        E3[l, j] = 1.0
        MSK[0, j] = 1.0
        for i in range(_NCOL):
            if i // 12 == h:
                HS[i, j] = 1.0
    for j in range(_NCOL, _NPAD):
        HS[j, j] = 1.0
    return cW, cH, cS, cHd, E3, HS, MSK


_CONSTS = _build_consts()

_UNPACK_PERM = np.concatenate(
    [h * 32 + np.concatenate([np.arange(16) * 2, np.arange(16) * 2 + 1])
     for h in range(_NH)]).astype(np.int32)


def _prep_body(hid_ref, enc_ref, rx_ref, ry_ref,
               wv_ref, bv_ref, wox_ref, box_ref, woy_ref, boy_ref,
               wat_ref, bat_ref, cw_ref, ch_ref, cs_ref, chd_ref, e3_ref,
               hs_ref, msk_ref,
               val_ref, attn_ref, ia_ref, ib_ref, ic_ref, id_ref,
               wa_ref, wb_ref, wc_ref, wd_ref):
    f32 = jnp.float32
    hp = lax.Precision.HIGHEST
    hid = hid_ref[...]
    enc = enc_ref[...]
    val = jnp.dot(enc, wv_ref[...], preferred_element_type=f32,
                  precision=hp) + bv_ref[...]
    val_ref[...] = val.astype(jnp.bfloat16)
    offx = jnp.dot(hid, wox_ref[...], preferred_element_type=f32,
                   precision=hp) + box_ref[...]
    offy = jnp.dot(hid, woy_ref[...], preferred_element_type=f32,
                   precision=hp) + boy_ref[...]
    logits = jnp.dot(hid, wat_ref[...], preferred_element_type=f32,
                     precision=hp) + bat_ref[...]
    m = jnp.max(logits, axis=1, keepdims=True)
    e = jnp.exp(logits - m)
    hsum = jnp.dot(e, hs_ref[...], preferred_element_type=f32, precision=hp)
    aw = e / hsum
    attn_ref[...] = aw[:, 0:_NCOL]

    cw = cw_ref[...]
    chh = ch_ref[...]
    rxe = jnp.dot(rx_ref[...], e3_ref[...], preferred_element_type=f32,
                  precision=hp)
    rye = jnp.dot(ry_ref[...], e3_ref[...], preferred_element_type=f32,
                  precision=hp)
    locx = rxe + offx / cw
    locy = rye + offy / chh
    gx = 2.0 * locx - 1.0
    gy = 2.0 * locy - 1.0
    px = ((gx + 1.0) * cw - 1.0) * 0.5
    py = ((gy + 1.0) * chh - 1.0) * 0.5
    x0 = jnp.floor(px)
    y0 = jnp.floor(py)
    x1 = x0 + 1.0
    y1 = y0 + 1.0
    wx0 = x1 - px
    wx1 = px - x0
    wy0 = y1 - py
    wy1 = py - y0
    vx0 = ((x0 >= 0.0) & (x0 <= cw - 1.0)).astype(f32)
    vx1 = ((x1 >= 0.0) & (x1 <= cw - 1.0)).astype(f32)
    vy0 = ((y0 >= 0.0) & (y0 <= chh - 1.0)).astype(f32)
    vy1 = ((y1 >= 0.0) & (y1 <= chh - 1.0)).astype(f32)
    ix0 = jnp.clip(x0, 0.0, cw - 1.0)
    ix1 = jnp.clip(x1, 0.0, cw - 1.0)
    iy0 = jnp.clip(y0, 0.0, chh - 1.0)
    iy1 = jnp.clip(y1, 0.0, chh - 1.0)
    cs = cs_ref[...]
    chd = chd_ref[...]
    msk = msk_ref[...]
    awm = aw * msk

    def rowidx(iy, ix):
        return ((cs + iy * cw + ix) * 8.0 + chd).astype(jnp.int32)

    ia_ref[...] = rowidx(iy0, ix0)
    ib_ref[...] = rowidx(iy1, ix0)
    ic_ref[...] = rowidx(iy0, ix1)
    id_ref[...] = rowidx(iy1, ix1)
    wa_ref[...] = wx0 * wy0 * vx0 * vy0 * awm
    wb_ref[...] = wx0 * wy1 * vx0 * vy1 * awm
    wc_ref[...] = wx1 * wy0 * vx1 * vy0 * awm
    wd_ref[...] = wx1 * wy1 * vx1 * vy1 * awm


def _row_block(i):
    return (i, 0)


def _full(i):
    return (0, 0)


_prep_call = pl.pallas_call(
    _prep_body,
    grid=(_Q // _BQ,),
    in_specs=[
        pl.BlockSpec((_BQ, 256), _row_block),
        pl.BlockSpec((_BQ, 256), _row_block),
        pl.BlockSpec((_BQ, 3), _row_block),
        pl.BlockSpec((_BQ, 3), _row_block),
        pl.BlockSpec((256, 256), _full),
        pl.BlockSpec((1, 256), _full),
        pl.BlockSpec((256, _NPAD), _full),
        pl.BlockSpec((1, _NPAD), _full),
        pl.BlockSpec((256, _NPAD), _full),
        pl.BlockSpec((1, _NPAD), _full),
        pl.BlockSpec((256, _NPAD), _full),
        pl.BlockSpec((1, _NPAD), _full),
        pl.BlockSpec((1, _NPAD), _full),
        pl.BlockSpec((1, _NPAD), _full),
        pl.BlockSpec((1, _NPAD), _full),
        pl.BlockSpec((1, _NPAD), _full),
        pl.BlockSpec((3, _NPAD), _full),
        pl.BlockSpec((_NPAD, _NPAD), _full),
        pl.BlockSpec((1, _NPAD), _full),
    ],
    out_specs=[
        pl.BlockSpec((_BQ, 256), _row_block),
        pl.BlockSpec((_BQ, _NCOL), _row_block),
        pl.BlockSpec((_BQ, _NPAD), _row_block),
        pl.BlockSpec((_BQ, _NPAD), _row_block),
        pl.BlockSpec((_BQ, _NPAD), _row_block),
        pl.BlockSpec((_BQ, _NPAD), _row_block),
        pl.BlockSpec((_BQ, _NPAD), _row_block),
        pl.BlockSpec((_BQ, _NPAD), _row_block),
        pl.BlockSpec((_BQ, _NPAD), _row_block),
        pl.BlockSpec((_BQ, _NPAD), _row_block),
    ],
    out_shape=[
        jax.ShapeDtypeStruct((_Q, 256), jnp.bfloat16),
        jax.ShapeDtypeStruct((_Q, _NCOL), jnp.float32),
        jax.ShapeDtypeStruct((_Q, _NPAD), jnp.int32),
        jax.ShapeDtypeStruct((_Q, _NPAD), jnp.int32),
        jax.ShapeDtypeStruct((_Q, _NPAD), jnp.int32),
        jax.ShapeDtypeStruct((_Q, _NPAD), jnp.int32),
        jax.ShapeDtypeStruct((_Q, _NPAD), jnp.float32),
        jax.ShapeDtypeStruct((_Q, _NPAD), jnp.float32),
        jax.ShapeDtypeStruct((_Q, _NPAD), jnp.float32),
        jax.ShapeDtypeStruct((_Q, _NPAD), jnp.float32),
    ],
)


def _outproj_body(x_ref, w_ref, b_ref, o_ref):
    o_ref[...] = jnp.dot(x_ref[...], w_ref[...],
                         preferred_element_type=jnp.float32,
                         precision=lax.Precision.HIGHEST) + b_ref[...]


_outproj_call = pl.pallas_call(
    _outproj_body,
    grid=(_Q // _BQ,),
    in_specs=[
        pl.BlockSpec((_BQ, 256), _row_block),
        pl.BlockSpec((256, 256), _full),
        pl.BlockSpec((1, 256), _full),
    ],
    out_specs=pl.BlockSpec((_BQ, 256), _row_block),
    out_shape=jax.ShapeDtypeStruct((_Q, 256), jnp.float32),
)


def _sc_body(table, ia, ib, ic, idd, wa, wb, wc, wd, out,
             ia_v, ib_v, ic_v, id_v, wa_v, wb_v, wc_v, wd_v,
             rows_v, out_v, sem0, sem1, sem2):
    wid = lax.axis_index("s") * 2 + lax.axis_index("c")
    base = wid * _QPW
    iv = (ia_v, ib_v, ic_v, id_v)
    wv = (wa_v, wb_v, wc_v, wd_v)
    ih = (ia, ib, ic, idd)
    wh = (wa, wb, wc, wd)
    sems = (sem0, sem1, sem2)

    def fire(ql, buf):
        for ci in range(4):
            pltpu.async_copy(table.at[iv[ci].at[ql, pl.ds(0, _NCOL)]],
                             rows_v.at[buf, pl.ds(ci * _NCOL, _NCOL)],
                             sems[buf])

    def drain(ql, buf):
        for ci in range(4):
            pltpu.make_async_copy(table.at[iv[ci].at[ql, pl.ds(0, _NCOL)]],
                                  rows_v.at[buf, pl.ds(ci * _NCOL, _NCOL)],
                                  sems[buf]).wait()

    def acc(ql, buf):
        wvecs = [[wv[ci][ql, pl.ds(k * 16, 16)] for k in range(6)]
                 for ci in range(4)]
        for h in range(8):
            a0 = jnp.zeros((16,), jnp.float32)
            a1 = jnp.zeros((16,), jnp.float32)
            for ci in range(4):
                for p in range(12):
                    col = h * 12 + p
                    w = wvecs[ci][col // 16][col % 16]
                    r = ci * _NCOL + col
                    lo, hi = plsc.unpack(
                        rows_v[buf, r, :], format=plsc.PackFormat.INTERLEAVED,
                        preferred_element_type=jnp.float32)
                    a0 = a0 + lo * w
                    a1 = a1 + hi * w
            out_v[ql, h * 32:h * 32 + 16] = a0
            out_v[ql, h * 32 + 16:h * 32 + 32] = a1

    @pl.loop(0, _NCHUNK)
    def _chunk(chunk):
        q0 = base + chunk * _QB
        for ci in range(4):
            pltpu.sync_copy(ih[ci].at[pl.ds(q0, _QB)], iv[ci])
            pltpu.sync_copy(wh[ci].at[pl.ds(q0, _QB)], wv[ci])
        fire(0, 0)
        fire(1, 1)
        fire(2, 2)

        @pl.loop(0, _QB, step=3)
        def _q(g):
            for b in range(3):
                drain(g + b, b)
                acc(g + b, b)

                @pl.when(g + b + 3 < _QB)
                def _(b=b):
                    fire(g + b + 3, b)

        pltpu.sync_copy(out_v, out.at[pl.ds(q0, _QB)])


@functools.cache
def _sc_call():
  return functools.partial(
    pl.kernel,
    out_type=jax.ShapeDtypeStruct((_Q, 256), jnp.float32),
    mesh=plsc.VectorSubcoreMesh(core_axis_name="c", subcore_axis_name="s"),
    compiler_params=pltpu.CompilerParams(use_tc_tiling_on_sc=False,
                                         needs_layout_passes=False),
    scratch_types=[
        pltpu.VMEM((_QB, _NPAD), jnp.int32),
        pltpu.VMEM((_QB, _NPAD), jnp.int32),
        pltpu.VMEM((_QB, _NPAD), jnp.int32),
        pltpu.VMEM((_QB, _NPAD), jnp.int32),
        pltpu.VMEM((_QB, _NPAD), jnp.float32),
        pltpu.VMEM((_QB, _NPAD), jnp.float32),
        pltpu.VMEM((_QB, _NPAD), jnp.float32),
        pltpu.VMEM((_QB, _NPAD), jnp.float32),
        pltpu.VMEM((3, 4 * _NCOL, _HD), jnp.bfloat16),
        pltpu.VMEM((_QB, 256), jnp.float32),
        pltpu.SemaphoreType.DMA,
        pltpu.SemaphoreType.DMA,
        pltpu.SemaphoreType.DMA,
    ],
  )(_sc_body)


def kernel(hidden_states, encoder_hidden_states, reference_points, W_value,
           b_value, W_off, b_off, W_attn, b_attn, W_out, b_out):
    hid = hidden_states[0]
    enc = encoder_hidden_states[0]
    rp = reference_points[0]
    rx = rp[:, :, 0]
    ry = rp[:, :, 1]
    pad = ((0, 0), (0, _NPAD - _NCOL))
    Wo = W_off.reshape(256, _NH, 12, 2)
    Wox = jnp.pad(Wo[..., 0].reshape(256, _NCOL), pad)
    Woy = jnp.pad(Wo[..., 1].reshape(256, _NCOL), pad)
    bo = b_off.reshape(_NH, 12, 2)
    box = jnp.pad(bo[..., 0].reshape(1, _NCOL), pad)
    boy = jnp.pad(bo[..., 1].reshape(1, _NCOL), pad)
    Wat = jnp.pad(W_attn, pad)
    bat = jnp.pad(b_attn.reshape(1, _NCOL), pad)
    cW, cH, cS, cHd, E3, HS, MSK = (jnp.asarray(c) for c in _CONSTS)

    (value, attn, ia, ib, ic, idd, wa, wb, wc, wd) = _prep_call(
        hid, enc, rx, ry, W_value, b_value.reshape(1, 256), Wox, box, Woy,
        boy, Wat, bat, cW, cH, cS, cHd, E3, HS, MSK)

    table = value.reshape(_Q * _NH, _HD)
    sampled = _sc_call()(table, ia, ib, ic, idd, wa, wb, wc, wd)
    W_out_p = W_out[jnp.asarray(_UNPACK_PERM)]
    out = _outproj_call(sampled, W_out_p, b_out.reshape(1, 256))
    return out[None], attn.reshape(1, _Q, _NH, 12)


# speedup vs baseline: 1.0667x; 1.0667x over previous
import functools

import numpy as np
import jax
import jax.numpy as jnp
from jax import lax
from jax.experimental import pallas as pl
from jax.experimental.pallas import tpu as pltpu
from jax.experimental.pallas import tpu_sc as plsc

_SPATIAL = ((128, 128), (64, 64), (32, 32))
_NH = 8
_HD = 32
_Q = 21504
_NCOL = 96
_NPAD = 128
_BQ = 512

_NW = 32
_QPW = _Q // _NW
_QB = 48
_NCHUNK = _QPW // _QB


def _build_consts():
    starts = (0, 16384, 20480)
    cW = np.ones((1, _NPAD), np.float32)
    cH = np.ones((1, _NPAD), np.float32)
    cS = np.zeros((1, _NPAD), np.float32)
    cHd = np.zeros((1, _NPAD), np.float32)
    E3 = np.zeros((3, _NPAD), np.float32)
    HS = np.zeros((_NPAD, _NPAD), np.float32)
    MSK = np.zeros((1, _NPAD), np.float32)
    for j in range(_NCOL):
        h = j // 12
        l = (j % 12) // 4
        hh, ww = _SPATIAL[l]
        cW[0, j] = ww
        cH[0, j] = hh
        cS[0, j] = starts[l]
        cHd[0, j] = h
        E3[l, j] = 1.0
        MSK[0, j] = 1.0
        for i in range(_NCOL):
            if i // 12 == h:
                HS[i, j] = 1.0
    for j in range(_NCOL, _NPAD):
        HS[j, j] = 1.0
    return cW, cH, cS, cHd, E3, HS, MSK


_CONSTS = _build_consts()

_UNPACK_PERM = np.concatenate(
    [h * 32 + np.concatenate([np.arange(16) * 2, np.arange(16) * 2 + 1])
     for h in range(_NH)]).astype(np.int32)


def _prep_body(hid_ref, enc_ref, rx_ref, ry_ref,
               wv_ref, bv_ref, wox_ref, box_ref, woy_ref, boy_ref,
               wat_ref, bat_ref, cw_ref, ch_ref, cs_ref, chd_ref, e3_ref,
               hs_ref, msk_ref,
               val_ref, attn_ref, ia_ref, ib_ref, ic_ref, id_ref,
               wa_ref, wb_ref, wc_ref, wd_ref):
    f32 = jnp.float32
    hp = lax.Precision.HIGHEST
    hid = hid_ref[...]
    enc = enc_ref[...]
    val = jnp.dot(enc, wv_ref[...], preferred_element_type=f32,
                  precision=hp) + bv_ref[...]
    val_ref[...] = val.astype(jnp.bfloat16)
    offx = jnp.dot(hid, wox_ref[...], preferred_element_type=f32,
                   precision=hp) + box_ref[...]
    offy = jnp.dot(hid, woy_ref[...], preferred_element_type=f32,
                   precision=hp) + boy_ref[...]
    logits = jnp.dot(hid, wat_ref[...], preferred_element_type=f32,
                     precision=hp) + bat_ref[...]
    m = jnp.max(logits, axis=1, keepdims=True)
    e = jnp.exp(logits - m)
    hsum = jnp.dot(e, hs_ref[...], preferred_element_type=f32, precision=hp)
    aw = e / hsum
    attn_ref[...] = aw[:, 0:_NCOL]

    cw = cw_ref[...]
    chh = ch_ref[...]
    rxe = jnp.dot(rx_ref[...], e3_ref[...], preferred_element_type=f32,
                  precision=hp)
    rye = jnp.dot(ry_ref[...], e3_ref[...], preferred_element_type=f32,
                  precision=hp)
    locx = rxe + offx / cw
    locy = rye + offy / chh
    gx = 2.0 * locx - 1.0
    gy = 2.0 * locy - 1.0
    px = ((gx + 1.0) * cw - 1.0) * 0.5
    py = ((gy + 1.0) * chh - 1.0) * 0.5
    x0 = jnp.floor(px)
    y0 = jnp.floor(py)
    x1 = x0 + 1.0
    y1 = y0 + 1.0
    wx0 = x1 - px
    wx1 = px - x0
    wy0 = y1 - py
    wy1 = py - y0
    vx0 = ((x0 >= 0.0) & (x0 <= cw - 1.0)).astype(f32)
    vx1 = ((x1 >= 0.0) & (x1 <= cw - 1.0)).astype(f32)
    vy0 = ((y0 >= 0.0) & (y0 <= chh - 1.0)).astype(f32)
    vy1 = ((y1 >= 0.0) & (y1 <= chh - 1.0)).astype(f32)
    ix0 = jnp.clip(x0, 0.0, cw - 1.0)
    ix1 = jnp.clip(x1, 0.0, cw - 1.0)
    iy0 = jnp.clip(y0, 0.0, chh - 1.0)
    iy1 = jnp.clip(y1, 0.0, chh - 1.0)
    cs = cs_ref[...]
    chd = chd_ref[...]
    msk = msk_ref[...]
    awm = aw * msk

    def rowidx(iy, ix):
        return ((cs + iy * cw + ix) * 8.0 + chd).astype(jnp.int32)

    ia_ref[...] = rowidx(iy0, ix0)
    ib_ref[...] = rowidx(iy1, ix0)
    ic_ref[...] = rowidx(iy0, ix1)
    id_ref[...] = rowidx(iy1, ix1)
    wa_ref[...] = wx0 * wy0 * vx0 * vy0 * awm
    wb_ref[...] = wx0 * wy1 * vx0 * vy1 * awm
    wc_ref[...] = wx1 * wy0 * vx1 * vy0 * awm
    wd_ref[...] = wx1 * wy1 * vx1 * vy1 * awm


def _row_block(i):
    return (i, 0)


def _full(i):
    return (0, 0)


_prep_call = pl.pallas_call(
    _prep_body,
    grid=(_Q // _BQ,),
    in_specs=[
        pl.BlockSpec((_BQ, 256), _row_block),
        pl.BlockSpec((_BQ, 256), _row_block),
        pl.BlockSpec((_BQ, 3), _row_block),
        pl.BlockSpec((_BQ, 3), _row_block),
        pl.BlockSpec((256, 256), _full),
        pl.BlockSpec((1, 256), _full),
        pl.BlockSpec((256, _NPAD), _full),
        pl.BlockSpec((1, _NPAD), _full),
        pl.BlockSpec((256, _NPAD), _full),
        pl.BlockSpec((1, _NPAD), _full),
        pl.BlockSpec((256, _NPAD), _full),
        pl.BlockSpec((1, _NPAD), _full),
        pl.BlockSpec((1, _NPAD), _full),
        pl.BlockSpec((1, _NPAD), _full),
        pl.BlockSpec((1, _NPAD), _full),
        pl.BlockSpec((1, _NPAD), _full),
        pl.BlockSpec((3, _NPAD), _full),
        pl.BlockSpec((_NPAD, _NPAD), _full),
        pl.BlockSpec((1, _NPAD), _full),
    ],
    out_specs=[
        pl.BlockSpec((_BQ, 256), _row_block),
        pl.BlockSpec((_BQ, _NCOL), _row_block),
        pl.BlockSpec((_BQ, _NPAD), _row_block),
        pl.BlockSpec((_BQ, _NPAD), _row_block),
        pl.BlockSpec((_BQ, _NPAD), _row_block),
        pl.BlockSpec((_BQ, _NPAD), _row_block),
        pl.BlockSpec((_BQ, _NPAD), _row_block),
        pl.BlockSpec((_BQ, _NPAD), _row_block),
        pl.BlockSpec((_BQ, _NPAD), _row_block),
        pl.BlockSpec((_BQ, _NPAD), _row_block),
    ],
    out_shape=[
        jax.ShapeDtypeStruct((_Q, 256), jnp.bfloat16),
        jax.ShapeDtypeStruct((_Q, _NCOL), jnp.float32),
        jax.ShapeDtypeStruct((_Q, _NPAD), jnp.int32),
        jax.ShapeDtypeStruct((_Q, _NPAD), jnp.int32),
        jax.ShapeDtypeStruct((_Q, _NPAD), jnp.int32),
        jax.ShapeDtypeStruct((_Q, _NPAD), jnp.int32),
        jax.ShapeDtypeStruct((_Q, _NPAD), jnp.float32),
        jax.ShapeDtypeStruct((_Q, _NPAD), jnp.float32),
        jax.ShapeDtypeStruct((_Q, _NPAD), jnp.float32),
        jax.ShapeDtypeStruct((_Q, _NPAD), jnp.float32),
    ],
)


def _outproj_body(x_ref, w_ref, b_ref, o_ref):
    o_ref[...] = jnp.dot(x_ref[...], w_ref[...],
                         preferred_element_type=jnp.float32,
                         precision=lax.Precision.HIGHEST) + b_ref[...]


_outproj_call = pl.pallas_call(
    _outproj_body,
    grid=(_Q // _BQ,),
    in_specs=[
        pl.BlockSpec((_BQ, 256), _row_block),
        pl.BlockSpec((256, 256), _full),
        pl.BlockSpec((1, 256), _full),
    ],
    out_specs=pl.BlockSpec((_BQ, 256), _row_block),
    out_shape=jax.ShapeDtypeStruct((_Q, 256), jnp.float32),
)


def _sc_body(table, ia, ib, ic, idd, wa, wb, wc, wd, out,
             ia_v, ib_v, ic_v, id_v, wa_v, wb_v, wc_v, wd_v,
             rows_v, out_v, sem0, sem1):
    wid = lax.axis_index("s") * 2 + lax.axis_index("c")
    base = wid * _QPW
    iv = (ia_v, ib_v, ic_v, id_v)
    wv = (wa_v, wb_v, wc_v, wd_v)
    ih = (ia, ib, ic, idd)
    wh = (wa, wb, wc, wd)
    sems = (sem0, sem1)

    def fire(ql, buf):
        for ci in range(4):
            pltpu.async_copy(table.at[iv[ci].at[ql, pl.ds(0, _NCOL)]],
                             rows_v.at[buf, pl.ds(ci * _NCOL, _NCOL)],
                             sems[buf])

    def drain(ql, buf):
        pltpu.make_async_copy(table.at[pl.ds(0, 4 * _NCOL)],
                              rows_v.at[buf], sems[buf]).wait()

    def acc(ql, buf):
        wvecs = [[wv[ci][ql, pl.ds(k * 16, 16)] for k in range(6)]
                 for ci in range(4)]
        for h in range(8):
            a0 = jnp.zeros((16,), jnp.float32)
            a1 = jnp.zeros((16,), jnp.float32)
            for ci in range(4):
                for p in range(12):
                    col = h * 12 + p
                    w = wvecs[ci][col // 16][col % 16]
                    r = ci * _NCOL + col
                    lo, hi = plsc.unpack(
                        rows_v[buf, r, :], format=plsc.PackFormat.INTERLEAVED,
                        preferred_element_type=jnp.float32)
                    a0 = a0 + lo * w
                    a1 = a1 + hi * w
            out_v[ql, h * 32:h * 32 + 16] = a0
            out_v[ql, h * 32 + 16:h * 32 + 32] = a1

    @pl.loop(0, _NCHUNK)
    def _chunk(chunk):
        q0 = base + chunk * _QB
        for ci in range(4):
            pltpu.sync_copy(ih[ci].at[pl.ds(q0, _QB)], iv[ci])
            pltpu.sync_copy(wh[ci].at[pl.ds(q0, _QB)], wv[ci])
        fire(0, 0)
        fire(1, 1)

        @pl.loop(0, _QB, step=2)
        def _q(g):
            drain(g, 0)
            acc(g, 0)

            @pl.when(g + 2 < _QB)
            def _():
                fire(g + 2, 0)

            drain(g + 1, 1)
            acc(g + 1, 1)

            @pl.when(g + 3 < _QB)
            def _():
                fire(g + 3, 1)

        pltpu.sync_copy(out_v, out.at[pl.ds(q0, _QB)])


@functools.cache
def _sc_call():
  return functools.partial(
    pl.kernel,
    out_type=jax.ShapeDtypeStruct((_Q, 256), jnp.float32),
    mesh=plsc.VectorSubcoreMesh(core_axis_name="c", subcore_axis_name="s"),
    compiler_params=pltpu.CompilerParams(use_tc_tiling_on_sc=False,
                                         needs_layout_passes=False),
    scratch_types=[
        pltpu.VMEM((_QB, _NPAD), jnp.int32),
        pltpu.VMEM((_QB, _NPAD), jnp.int32),
        pltpu.VMEM((_QB, _NPAD), jnp.int32),
        pltpu.VMEM((_QB, _NPAD), jnp.int32),
        pltpu.VMEM((_QB, _NPAD), jnp.float32),
        pltpu.VMEM((_QB, _NPAD), jnp.float32),
        pltpu.VMEM((_QB, _NPAD), jnp.float32),
        pltpu.VMEM((_QB, _NPAD), jnp.float32),
        pltpu.VMEM((2, 4 * _NCOL, _HD), jnp.bfloat16),
        pltpu.VMEM((_QB, 256), jnp.float32),
        pltpu.SemaphoreType.DMA,
        pltpu.SemaphoreType.DMA,
    ],
  )(_sc_body)


def kernel(hidden_states, encoder_hidden_states, reference_points, W_value,
           b_value, W_off, b_off, W_attn, b_attn, W_out, b_out):
    hid = hidden_states[0]
    enc = encoder_hidden_states[0]
    rp = reference_points[0]
    rx = rp[:, :, 0]
    ry = rp[:, :, 1]
    pad = ((0, 0), (0, _NPAD - _NCOL))
    Wo = W_off.reshape(256, _NH, 12, 2)
    Wox = jnp.pad(Wo[..., 0].reshape(256, _NCOL), pad)
    Woy = jnp.pad(Wo[..., 1].reshape(256, _NCOL), pad)
    bo = b_off.reshape(_NH, 12, 2)
    box = jnp.pad(bo[..., 0].reshape(1, _NCOL), pad)
    boy = jnp.pad(bo[..., 1].reshape(1, _NCOL), pad)
    Wat = jnp.pad(W_attn, pad)
    bat = jnp.pad(b_attn.reshape(1, _NCOL), pad)
    cW, cH, cS, cHd, E3, HS, MSK = (jnp.asarray(c) for c in _CONSTS)

    (value, attn, ia, ib, ic, idd, wa, wb, wc, wd) = _prep_call(
        hid, enc, rx, ry, W_value, b_value.reshape(1, 256), Wox, box, Woy,
        boy, Wat, bat, cW, cH, cS, cHd, E3, HS, MSK)

    table = value.reshape(_Q * _NH, _HD)
    sampled = _sc_call()(table, ia, ib, ic, idd, wa, wb, wc, wd)
    W_out_p = W_out[jnp.asarray(_UNPACK_PERM)]
    out = _outproj_call(sampled, W_out_p, b_out.reshape(1, 256))
    return out[None], attn.reshape(1, _Q, _NH, 12)


# speedup vs baseline: 1.1235x; 1.0532x over previous
import functools

import numpy as np
import jax
import jax.numpy as jnp
from jax import lax
from jax.experimental import pallas as pl
from jax.experimental.pallas import tpu as pltpu
from jax.experimental.pallas import tpu_sc as plsc

_SPATIAL = ((128, 128), (64, 64), (32, 32))
_NH = 8
_HD = 32
_Q = 21504
_NCOL = 96
_NPAD = 128
_BQ = 512

_NW = 32
_QPW = _Q // _NW
_QB = 48
_NCHUNK = _QPW // _QB


def _build_consts():
    starts = (0, 16384, 20480)
    cW = np.ones((1, _NPAD), np.float32)
    cH = np.ones((1, _NPAD), np.float32)
    cS = np.zeros((1, _NPAD), np.float32)
    cHd = np.zeros((1, _NPAD), np.float32)
    E3 = np.zeros((3, _NPAD), np.float32)
    HS = np.zeros((_NPAD, _NPAD), np.float32)
    MSK = np.zeros((1, _NPAD), np.float32)
    for j in range(_NCOL):
        h = j // 12
        l = (j % 12) // 4
        hh, ww = _SPATIAL[l]
        cW[0, j] = ww
        cH[0, j] = hh
        cS[0, j] = starts[l]
        cHd[0, j] = h
        E3[l, j] = 1.0
        MSK[0, j] = 1.0
        for i in range(_NCOL):
            if i // 12 == h:
                HS[i, j] = 1.0
    for j in range(_NCOL, _NPAD):
        HS[j, j] = 1.0
    return cW, cH, cS, cHd, E3, HS, MSK


_CONSTS = _build_consts()

_UNPACK_PERM = np.concatenate(
    [h * 32 + np.concatenate([np.arange(16) * 2, np.arange(16) * 2 + 1])
     for h in range(_NH)]).astype(np.int32)


def _prep_body(hid_ref, enc_ref, rx_ref, ry_ref,
               wv_ref, bv_ref, wox_ref, box_ref, woy_ref, boy_ref,
               wat_ref, bat_ref, cw_ref, ch_ref, cs_ref, chd_ref, e3_ref,
               hs_ref, msk_ref,
               val_ref, attn_ref, ia_ref, ib_ref, ic_ref, id_ref,
               wa_ref, wb_ref, wc_ref, wd_ref):
    f32 = jnp.float32
    hp = lax.Precision.HIGHEST
    hid = hid_ref[...]
    enc = enc_ref[...]
    val_ref[...] = jnp.dot(enc, wv_ref[...], preferred_element_type=f32,
                           precision=hp) + bv_ref[...]
    offx = jnp.dot(hid, wox_ref[...], preferred_element_type=f32,
                   precision=hp) + box_ref[...]
    offy = jnp.dot(hid, woy_ref[...], preferred_element_type=f32,
                   precision=hp) + boy_ref[...]
    logits = jnp.dot(hid, wat_ref[...], preferred_element_type=f32,
                     precision=hp) + bat_ref[...]
    m = jnp.max(logits, axis=1, keepdims=True)
    e = jnp.exp(logits - m)
    hsum = jnp.dot(e, hs_ref[...], preferred_element_type=f32, precision=hp)
    aw = e / hsum
    attn_ref[...] = aw[:, 0:_NCOL]

    cw = cw_ref[...]
    chh = ch_ref[...]
    rxe = jnp.dot(rx_ref[...], e3_ref[...], preferred_element_type=f32,
                  precision=hp)
    rye = jnp.dot(ry_ref[...], e3_ref[...], preferred_element_type=f32,
                  precision=hp)
    locx = rxe + offx / cw
    locy = rye + offy / chh
    gx = 2.0 * locx - 1.0
    gy = 2.0 * locy - 1.0
    px = ((gx + 1.0) * cw - 1.0) * 0.5
    py = ((gy + 1.0) * chh - 1.0) * 0.5
    x0 = jnp.floor(px)
    y0 = jnp.floor(py)
    x1 = x0 + 1.0
    y1 = y0 + 1.0
    wx0 = x1 - px
    wx1 = px - x0
    wy0 = y1 - py
    wy1 = py - y0
    vx0 = ((x0 >= 0.0) & (x0 <= cw - 1.0)).astype(f32)
    vx1 = ((x1 >= 0.0) & (x1 <= cw - 1.0)).astype(f32)
    vy0 = ((y0 >= 0.0) & (y0 <= chh - 1.0)).astype(f32)
    vy1 = ((y1 >= 0.0) & (y1 <= chh - 1.0)).astype(f32)
    ix0 = jnp.clip(x0, 0.0, cw - 1.0)
    ix1 = jnp.clip(x1, 0.0, cw - 1.0)
    iy0 = jnp.clip(y0, 0.0, chh - 1.0)
    iy1 = jnp.clip(y1, 0.0, chh - 1.0)
    cs = cs_ref[...]
    chd = chd_ref[...]
    msk = msk_ref[...]
    awm = aw * msk

    def rowidx(iy, ix):
        return ((cs + iy * cw + ix) * 8.0 + chd).astype(jnp.int32)

    ia_ref[...] = rowidx(iy0, ix0)
    ib_ref[...] = rowidx(iy1, ix0)
    ic_ref[...] = rowidx(iy0, ix1)
    id_ref[...] = rowidx(iy1, ix1)
    wa_ref[...] = wx0 * wy0 * vx0 * vy0 * awm
    wb_ref[...] = wx0 * wy1 * vx0 * vy1 * awm
    wc_ref[...] = wx1 * wy0 * vx1 * vy0 * awm
    wd_ref[...] = wx1 * wy1 * vx1 * vy1 * awm


def _row_block(i):
    return (i, 0)


def _full(i):
    return (0, 0)


_prep_call = pl.pallas_call(
    _prep_body,
    grid=(_Q // _BQ,),
    in_specs=[
        pl.BlockSpec((_BQ, 256), _row_block),
        pl.BlockSpec((_BQ, 256), _row_block),
        pl.BlockSpec((_BQ, 3), _row_block),
        pl.BlockSpec((_BQ, 3), _row_block),
        pl.BlockSpec((256, 256), _full),
        pl.BlockSpec((1, 256), _full),
        pl.BlockSpec((256, _NPAD), _full),
        pl.BlockSpec((1, _NPAD), _full),
        pl.BlockSpec((256, _NPAD), _full),
        pl.BlockSpec((1, _NPAD), _full),
        pl.BlockSpec((256, _NPAD), _full),
        pl.BlockSpec((1, _NPAD), _full),
        pl.BlockSpec((1, _NPAD), _full),
        pl.BlockSpec((1, _NPAD), _full),
        pl.BlockSpec((1, _NPAD), _full),
        pl.BlockSpec((1, _NPAD), _full),
        pl.BlockSpec((3, _NPAD), _full),
        pl.BlockSpec((_NPAD, _NPAD), _full),
        pl.BlockSpec((1, _NPAD), _full),
    ],
    out_specs=[
        pl.BlockSpec((_BQ, 256), _row_block),
        pl.BlockSpec((_BQ, _NCOL), _row_block),
        pl.BlockSpec((_BQ, _NPAD), _row_block),
        pl.BlockSpec((_BQ, _NPAD), _row_block),
        pl.BlockSpec((_BQ, _NPAD), _row_block),
        pl.BlockSpec((_BQ, _NPAD), _row_block),
        pl.BlockSpec((_BQ, _NPAD), _row_block),
        pl.BlockSpec((_BQ, _NPAD), _row_block),
        pl.BlockSpec((_BQ, _NPAD), _row_block),
        pl.BlockSpec((_BQ, _NPAD), _row_block),
    ],
    out_shape=[
        jax.ShapeDtypeStruct((_Q, 256), jnp.float32),
        jax.ShapeDtypeStruct((_Q, _NCOL), jnp.float32),
        jax.ShapeDtypeStruct((_Q, _NPAD), jnp.int32),
        jax.ShapeDtypeStruct((_Q, _NPAD), jnp.int32),
        jax.ShapeDtypeStruct((_Q, _NPAD), jnp.int32),
        jax.ShapeDtypeStruct((_Q, _NPAD), jnp.int32),
        jax.ShapeDtypeStruct((_Q, _NPAD), jnp.float32),
        jax.ShapeDtypeStruct((_Q, _NPAD), jnp.float32),
        jax.ShapeDtypeStruct((_Q, _NPAD), jnp.float32),
        jax.ShapeDtypeStruct((_Q, _NPAD), jnp.float32),
    ],
)


def _outproj_body(x_ref, w_ref, b_ref, o_ref):
    o_ref[...] = jnp.dot(x_ref[...], w_ref[...],
                         preferred_element_type=jnp.float32,
                         precision=lax.Precision.HIGHEST) + b_ref[...]


_outproj_call = pl.pallas_call(
    _outproj_body,
    grid=(_Q // _BQ,),
    in_specs=[
        pl.BlockSpec((_BQ, 256), _row_block),
        pl.BlockSpec((256, 256), _full),
        pl.BlockSpec((1, 256), _full),
    ],
    out_specs=pl.BlockSpec((_BQ, 256), _row_block),
    out_shape=jax.ShapeDtypeStruct((_Q, 256), jnp.float32),
)


def _sc_body(table, ia, ib, ic, idd, wa, wb, wc, wd, out,
             ia_v, ib_v, ic_v, id_v, wa_v, wb_v, wc_v, wd_v,
             rows_v, out_v, sem0, sem1):
    wid = lax.axis_index("s") * 2 + lax.axis_index("c")
    base = wid * _QPW
    iv = (ia_v, ib_v, ic_v, id_v)
    wv = (wa_v, wb_v, wc_v, wd_v)
    ih = (ia, ib, ic, idd)
    wh = (wa, wb, wc, wd)
    sems = (sem0, sem1)

    def fire(ql, buf):
        for ci in range(4):
            pltpu.async_copy(table.at[iv[ci].at[ql, pl.ds(0, _NCOL)]],
                             rows_v.at[buf, pl.ds(ci * _NCOL, _NCOL)],
                             sems[buf])

    def drain(ql, buf):
        pltpu.make_async_copy(table.at[pl.ds(0, 4 * _NCOL)],
                              rows_v.at[buf], sems[buf]).wait()

    def acc(ql, buf):
        wvecs = [[wv[ci][ql, pl.ds(k * 16, 16)] for k in range(6)]
                 for ci in range(4)]
        for h in range(8):
            a0 = jnp.zeros((16,), jnp.float32)
            a1 = jnp.zeros((16,), jnp.float32)
            for ci in range(4):
                for p in range(12):
                    col = h * 12 + p
                    w = wvecs[ci][col // 16][col % 16]
                    r = ci * _NCOL + col
                    a0 = a0 + rows_v[buf, r, 0:16] * w
                    a1 = a1 + rows_v[buf, r, 16:32] * w
            out_v[ql, h * 32:h * 32 + 16] = a0
            out_v[ql, h * 32 + 16:h * 32 + 32] = a1

    @pl.loop(0, _NCHUNK)
    def _chunk(chunk):
        q0 = base + chunk * _QB
        for ci in range(4):
            pltpu.sync_copy(ih[ci].at[pl.ds(q0, _QB)], iv[ci])
            pltpu.sync_copy(wh[ci].at[pl.ds(q0, _QB)], wv[ci])
        fire(0, 0)
        fire(1, 1)

        @pl.loop(0, _QB, step=2)
        def _q(g):
            drain(g, 0)
            acc(g, 0)

            @pl.when(g + 2 < _QB)
            def _():
                fire(g + 2, 0)

            drain(g + 1, 1)
            acc(g + 1, 1)

            @pl.when(g + 3 < _QB)
            def _():
                fire(g + 3, 1)

        pltpu.sync_copy(out_v, out.at[pl.ds(q0, _QB)])


@functools.cache
def _sc_call():
  return functools.partial(
    pl.kernel,
    out_type=jax.ShapeDtypeStruct((_Q, 256), jnp.float32),
    mesh=plsc.VectorSubcoreMesh(core_axis_name="c", subcore_axis_name="s"),
    compiler_params=pltpu.CompilerParams(use_tc_tiling_on_sc=False,
                                         needs_layout_passes=False),
    scratch_types=[
        pltpu.VMEM((_QB, _NPAD), jnp.int32),
        pltpu.VMEM((_QB, _NPAD), jnp.int32),
        pltpu.VMEM((_QB, _NPAD), jnp.int32),
        pltpu.VMEM((_QB, _NPAD), jnp.int32),
        pltpu.VMEM((_QB, _NPAD), jnp.float32),
        pltpu.VMEM((_QB, _NPAD), jnp.float32),
        pltpu.VMEM((_QB, _NPAD), jnp.float32),
        pltpu.VMEM((_QB, _NPAD), jnp.float32),
        pltpu.VMEM((2, 4 * _NCOL, _HD), jnp.float32),
        pltpu.VMEM((_QB, 256), jnp.float32),
        pltpu.SemaphoreType.DMA,
        pltpu.SemaphoreType.DMA,
    ],
  )(_sc_body)


def kernel(hidden_states, encoder_hidden_states, reference_points, W_value,
           b_value, W_off, b_off, W_attn, b_attn, W_out, b_out):
    hid = hidden_states[0]
    enc = encoder_hidden_states[0]
    rp = reference_points[0]
    rx = rp[:, :, 0]
    ry = rp[:, :, 1]
    pad = ((0, 0), (0, _NPAD - _NCOL))
    Wo = W_off.reshape(256, _NH, 12, 2)
    Wox = jnp.pad(Wo[..., 0].reshape(256, _NCOL), pad)
    Woy = jnp.pad(Wo[..., 1].reshape(256, _NCOL), pad)
    bo = b_off.reshape(_NH, 12, 2)
    box = jnp.pad(bo[..., 0].reshape(1, _NCOL), pad)
    boy = jnp.pad(bo[..., 1].reshape(1, _NCOL), pad)
    Wat = jnp.pad(W_attn, pad)
    bat = jnp.pad(b_attn.reshape(1, _NCOL), pad)
    cW, cH, cS, cHd, E3, HS, MSK = (jnp.asarray(c) for c in _CONSTS)

    (value, attn, ia, ib, ic, idd, wa, wb, wc, wd) = _prep_call(
        hid, enc, rx, ry, W_value, b_value.reshape(1, 256), Wox, box, Woy,
        boy, Wat, bat, cW, cH, cS, cHd, E3, HS, MSK)

    table = value.reshape(_Q * _NH, _HD)
    sampled = _sc_call()(table, ia, ib, ic, idd, wa, wb, wc, wd)
    out = _outproj_call(sampled, W_out, b_out.reshape(1, 256))
    return out[None], attn.reshape(1, _Q, _NH, 12)
